# baseline (speedup 1.0000x reference)
import functools

import jax
import jax.numpy as jnp
from jax import lax
from jax.experimental import pallas as pl
from jax.experimental.pallas import tpu as pltpu
from jax.experimental.pallas import tpu_sc as plsc

_LANES = 16
_EMB = 128
_IDX_PAD = 8


def _sc_body(u_idx_hbm, v_idx_hbm, u_hbm, v_hbm, out_hbm,
             u_idx_v, v_idx_v, u_rows, v_rows, out_v, sem):
    first = jnp.logical_and(lax.axis_index("c") == 0, lax.axis_index("s") == 0)

    @pl.when(first)
    def _():
        pltpu.sync_copy(u_idx_hbm, u_idx_v)
        pltpu.sync_copy(v_idx_hbm, v_idx_v)
        cu = pltpu.async_copy(u_hbm.at[u_idx_v], u_rows, sem)
        cv = pltpu.async_copy(v_hbm.at[v_idx_v], v_rows, sem)
        cu.wait()
        cv.wait()

        acc_p = jnp.zeros((_LANES,), jnp.float32)
        acc_n = jnp.zeros((_LANES,), jnp.float32)
        for j in range(_EMB // _LANES):
            sl = pl.ds(j * _LANES, _LANES)
            acc_p = acc_p + u_rows[0, sl] * v_rows[0, sl]
            acc_n = acc_n + u_rows[1, sl] * v_rows[1, sl]

        lane = lax.iota(jnp.int32, _LANES)

        def _shuffle(x, idx):
            return lax.gather(
                x, idx[:, None],
                dimension_numbers=lax.GatherDimensionNumbers(
                    offset_dims=(), collapsed_slice_dims=(0,),
                    start_index_map=(0,)),
                slice_sizes=(1,),
                mode=lax.GatherScatterMode.PROMISE_IN_BOUNDS)

        def _lane_sum(x):
            for s in (8, 4, 2, 1):
                idx = jnp.bitwise_and(lane + s, _LANES - 1)
                x = x + _shuffle(x, idx)
            return x

        d_pos = _lane_sum(acc_p)
        d_neg = _lane_sum(acc_n)

        a = jnp.where(lane == 0, d_pos, -d_neg)
        t = jnp.exp(-jnp.abs(a))
        z = 1.0 + t
        log1p_t = 2.0 * t / (2.0 + t)
        for _ in range(3):
            log1p_t = log1p_t + z * jnp.exp(-log1p_t) - 1.0
        y = jnp.maximum(-a, 0.0) + log1p_t

        y = jnp.where(lane < 2, y, 0.0)
        y_shift = _shuffle(y, jnp.bitwise_and(lane + 1, _LANES - 1))
        out_v[...] = y + y_shift
        pltpu.sync_copy(out_v, out_hbm)


def kernel(target_context_pos_word_id_pair, target_context_neg_word_id_pair,
           u_embeddings, v_embeddings):
    pos = target_context_pos_word_id_pair.astype(jnp.int32)
    neg = target_context_neg_word_id_pair.astype(jnp.int32)
    u_idx = jnp.pad(jnp.stack([pos[0], neg[0]]), (0, _IDX_PAD - 2))
    v_idx = jnp.pad(jnp.stack([pos[1], neg[1]]), (0, _IDX_PAD - 2))

    mesh = plsc.VectorSubcoreMesh(core_axis_name="c", subcore_axis_name="s")
    run = functools.partial(
        pl.kernel,
        mesh=mesh,
        out_type=jax.ShapeDtypeStruct((_LANES,), jnp.float32),
        scratch_types=[
            pltpu.VMEM((_IDX_PAD,), jnp.int32),
            pltpu.VMEM((_IDX_PAD,), jnp.int32),
            pltpu.VMEM((_IDX_PAD, _EMB), jnp.float32),
            pltpu.VMEM((_IDX_PAD, _EMB), jnp.float32),
            pltpu.VMEM((_LANES,), jnp.float32),
            pltpu.SemaphoreType.DMA,
        ],
    )(_sc_body)
    out = run(u_idx, v_idx, u_embeddings, v_embeddings)
    return out[0:1]


# speedup vs baseline: 1.1903x; 1.1903x over previous
import functools

import jax
import jax.numpy as jnp
from jax import lax
from jax.experimental import pallas as pl
from jax.experimental.pallas import tpu as pltpu
from jax.experimental.pallas import tpu_sc as plsc

_LANES = 16
_EMB = 128


def _sc_body(pos_hbm, neg_hbm, u_hbm, v_hbm, out_hbm,
             pos_v, neg_v, u_pos, v_pos, u_neg, v_neg, out_v, sem, gsem):
    first = jnp.logical_and(lax.axis_index("c") == 0, lax.axis_index("s") == 0)

    @pl.when(first)
    def _():
        cp = pltpu.async_copy(pos_hbm, pos_v, sem)
        cn = pltpu.async_copy(neg_hbm, neg_v, sem)
        cp.wait()
        cn.wait()
        g1 = pltpu.async_copy(u_hbm.at[pos_v], u_pos, gsem)
        g2 = pltpu.async_copy(v_hbm.at[pos_v], v_pos, gsem)
        g3 = pltpu.async_copy(u_hbm.at[neg_v], u_neg, gsem)
        g4 = pltpu.async_copy(v_hbm.at[neg_v], v_neg, gsem)
        g1.wait()
        g2.wait()
        g3.wait()
        g4.wait()

        acc_p = jnp.zeros((_LANES,), jnp.float32)
        acc_n = jnp.zeros((_LANES,), jnp.float32)
        for j in range(_EMB // _LANES):
            sl = pl.ds(j * _LANES, _LANES)
            acc_p = acc_p + u_pos[0, sl] * v_pos[1, sl]
            acc_n = acc_n + u_neg[0, sl] * v_neg[1, sl]

        lane = lax.iota(jnp.int32, _LANES)

        def _shuffle(x, idx):
            return lax.gather(
                x, idx[:, None],
                dimension_numbers=lax.GatherDimensionNumbers(
                    offset_dims=(), collapsed_slice_dims=(0,),
                    start_index_map=(0,)),
                slice_sizes=(1,),
                mode=lax.GatherScatterMode.PROMISE_IN_BOUNDS)

        def _lane_sum(x):
            for s in (8, 4, 2, 1):
                idx = jnp.bitwise_and(lane + s, _LANES - 1)
                x = x + _shuffle(x, idx)
            return x

        d_pos = _lane_sum(acc_p)
        d_neg = _lane_sum(acc_n)

        a = jnp.where(lane == 0, d_pos, -d_neg)
        t = jnp.exp(-jnp.abs(a))
        z = 1.0 + t
        log1p_t = 2.0 * t / (2.0 + t)
        for _ in range(3):
            log1p_t = log1p_t + z * jnp.exp(-log1p_t) - 1.0
        y = jnp.maximum(-a, 0.0) + log1p_t

        y = jnp.where(lane < 2, y, 0.0)
        y_shift = _shuffle(y, jnp.bitwise_and(lane + 1, _LANES - 1))
        out_v[...] = y + y_shift
        pltpu.sync_copy(out_v.at[pl.ds(0, 1)], out_hbm)


def kernel(target_context_pos_word_id_pair, target_context_neg_word_id_pair,
           u_embeddings, v_embeddings):
    pos = target_context_pos_word_id_pair.astype(jnp.int32)
    neg = target_context_neg_word_id_pair.astype(jnp.int32)

    mesh = plsc.VectorSubcoreMesh(core_axis_name="c", subcore_axis_name="s")
    run = functools.partial(
        pl.kernel,
        mesh=mesh,
        out_type=jax.ShapeDtypeStruct((1,), jnp.float32),
        scratch_types=[
            pltpu.VMEM((2,), jnp.int32),
            pltpu.VMEM((2,), jnp.int32),
            pltpu.VMEM((2, _EMB), jnp.float32),
            pltpu.VMEM((2, _EMB), jnp.float32),
            pltpu.VMEM((2, _EMB), jnp.float32),
            pltpu.VMEM((2, _EMB), jnp.float32),
            pltpu.VMEM((_LANES,), jnp.float32),
            pltpu.SemaphoreType.DMA,
            pltpu.SemaphoreType.DMA,
        ],
    )(_sc_body)
    return run(pos, neg, u_embeddings, v_embeddings)


# speedup vs baseline: 1.2904x; 1.0840x over previous
import functools

import jax
import jax.numpy as jnp
from jax import lax
from jax.experimental import pallas as pl
from jax.experimental.pallas import tpu as pltpu
from jax.experimental.pallas import tpu_sc as plsc

_LANES = 16
_EMB = 128


def _sc_body(pos_hbm, neg_hbm, u_hbm, v_hbm, out_hbm,
             pos_v, neg_v, u_pos, v_pos, u_neg, v_neg, out_v, sem, gsem):
    first = jnp.logical_and(lax.axis_index("c") == 0, lax.axis_index("s") == 0)

    @pl.when(first)
    def _():
        cp = pltpu.async_copy(pos_hbm, pos_v, sem)
        cn = pltpu.async_copy(neg_hbm, neg_v, sem)
        cp.wait()
        cn.wait()
        g1 = pltpu.async_copy(u_hbm.at[pos_v], u_pos, gsem)
        g2 = pltpu.async_copy(v_hbm.at[pos_v], v_pos, gsem)
        g3 = pltpu.async_copy(u_hbm.at[neg_v], u_neg, gsem)
        g4 = pltpu.async_copy(v_hbm.at[neg_v], v_neg, gsem)
        g1.wait()
        g2.wait()
        g3.wait()
        g4.wait()

        acc_p = jnp.zeros((_LANES,), jnp.float32)
        acc_n = jnp.zeros((_LANES,), jnp.float32)
        for j in range(_EMB // _LANES):
            sl = pl.ds(j * _LANES, _LANES)
            acc_p = acc_p + u_pos[0, sl] * v_pos[1, sl]
            acc_n = acc_n + u_neg[0, sl] * v_neg[1, sl]

        lane = lax.iota(jnp.int32, _LANES)

        def _shuffle(x, idx):
            return lax.gather(
                x, idx[:, None],
                dimension_numbers=lax.GatherDimensionNumbers(
                    offset_dims=(), collapsed_slice_dims=(0,),
                    start_index_map=(0,)),
                slice_sizes=(1,),
                mode=lax.GatherScatterMode.PROMISE_IN_BOUNDS)

        def _lane_sum(x):
            for s in (8, 4, 2, 1):
                idx = jnp.bitwise_and(lane + s, _LANES - 1)
                x = x + _shuffle(x, idx)
            return x

        d_pos = _lane_sum(acc_p)
        d_neg = _lane_sum(acc_n)

        a = jnp.where(lane == 0, d_pos, -d_neg)
        t = jnp.exp(-jnp.abs(a))
        z = 1.0 + t
        log1p_t = 2.0 * t / (2.0 + t)
        for _ in range(3):
            log1p_t = log1p_t + z * jnp.exp(-log1p_t) - 1.0
        y = jnp.maximum(-a, 0.0) + log1p_t

        y = jnp.where(lane < 2, y, 0.0)
        y_shift = _shuffle(y, jnp.bitwise_and(lane + 1, _LANES - 1))
        out_v[...] = y + y_shift
        pltpu.sync_copy(out_v.at[pl.ds(0, 1)], out_hbm)


def kernel(target_context_pos_word_id_pair, target_context_neg_word_id_pair,
           u_embeddings, v_embeddings):
    pos = target_context_pos_word_id_pair.astype(jnp.int32)
    neg = target_context_neg_word_id_pair.astype(jnp.int32)

    mesh = plsc.VectorSubcoreMesh(core_axis_name="c", subcore_axis_name="s",
                                  num_cores=1)
    run = functools.partial(
        pl.kernel,
        mesh=mesh,
        out_type=jax.ShapeDtypeStruct((1,), jnp.float32),
        scratch_types=[
            pltpu.VMEM((2,), jnp.int32),
            pltpu.VMEM((2,), jnp.int32),
            pltpu.VMEM((2, _EMB), jnp.float32),
            pltpu.VMEM((2, _EMB), jnp.float32),
            pltpu.VMEM((2, _EMB), jnp.float32),
            pltpu.VMEM((2, _EMB), jnp.float32),
            pltpu.VMEM((_LANES,), jnp.float32),
            pltpu.SemaphoreType.DMA,
            pltpu.SemaphoreType.DMA,
        ],
    )(_sc_body)
    return run(pos, neg, u_embeddings, v_embeddings)
